# baseline (speedup 1.0000x reference)
import functools

import jax
import jax.numpy as jnp
from jax import lax
from jax.experimental import pallas as pl
from jax.experimental.pallas import tpu as pltpu
from jax.experimental.pallas import tpu_sc as plsc

BATCH = 16384
EMBED_DIM = 64
LANES = 16
NUM_WORKERS = 32
B_PER_W = BATCH // NUM_WORKERS
GROUPS = B_PER_W // LANES


def _mf_body(users_hbm, items_hbm, utab_hbm, itab_hbm, out_hbm,
             uidx_v, iidx_v, urows_v, irows_v, out_v, part_v, usem, isem):
    wid = lax.axis_index("s") * 2 + lax.axis_index("c")
    base = wid * B_PER_W

    pltpu.sync_copy(users_hbm.at[pl.ds(base, B_PER_W)], uidx_v)
    pltpu.sync_copy(items_hbm.at[pl.ds(base, B_PER_W)], iidx_v)

    ucp = pltpu.async_copy(utab_hbm.at[uidx_v], urows_v, usem)
    icp = pltpu.async_copy(itab_hbm.at[iidx_v], irows_v, isem)
    ucp.wait()
    icp.wait()

    lane_iota = lax.iota(jnp.int32, LANES)

    def group(g, carry):
        rowbase = g * LANES
        for r in range(LANES):
            row = rowbase + r
            acc = (urows_v[row, pl.ds(0, LANES)] * irows_v[row, pl.ds(0, LANES)])
            for k in range(1, EMBED_DIM // LANES):
                acc = acc + (urows_v[row, pl.ds(k * LANES, LANES)]
                             * irows_v[row, pl.ds(k * LANES, LANES)])
            part_v[pl.ds(r * LANES, LANES)] = acc
        tot = plsc.load_gather(part_v, [lane_iota * LANES])
        for l in range(1, LANES):
            tot = tot + plsc.load_gather(part_v, [lane_iota * LANES + l])
        out_v[pl.ds(g * LANES, LANES)] = 1.0 / (1.0 + jnp.exp(-tot))
        return carry

    lax.fori_loop(0, GROUPS, group, 0)

    pltpu.sync_copy(out_v, out_hbm.at[pl.ds(base, B_PER_W)])


@jax.jit
def _mf(users, items, user_table, item_table):
    mesh = plsc.VectorSubcoreMesh(core_axis_name="c", subcore_axis_name="s")
    run = functools.partial(
        pl.kernel,
        mesh=mesh,
        compiler_params=pltpu.CompilerParams(needs_layout_passes=False,
                                             use_tc_tiling_on_sc=False),
        out_type=jax.ShapeDtypeStruct((BATCH,), jnp.float32),
        scratch_types=[
            pltpu.VMEM((B_PER_W,), jnp.int32),
            pltpu.VMEM((B_PER_W,), jnp.int32),
            pltpu.VMEM((B_PER_W, EMBED_DIM), jnp.float32),
            pltpu.VMEM((B_PER_W, EMBED_DIM), jnp.float32),
            pltpu.VMEM((B_PER_W,), jnp.float32),
            pltpu.VMEM((LANES * LANES,), jnp.float32),
            pltpu.SemaphoreType.DMA,
            pltpu.SemaphoreType.DMA,
        ],
    )(_mf_body)
    return run(users, items, user_table, item_table)


def kernel(users, items, user_table, item_table):
    return _mf(users.astype(jnp.int32), items.astype(jnp.int32),
               user_table, item_table)


# speedup vs baseline: 1.5478x; 1.5478x over previous
import functools

import jax
import jax.numpy as jnp
from jax import lax
from jax.experimental import pallas as pl
from jax.experimental.pallas import tpu as pltpu
from jax.experimental.pallas import tpu_sc as plsc

BATCH = 16384
EMBED_DIM = 64
LANES = 16
NUM_WORKERS = 32
B_PER_W = BATCH // NUM_WORKERS
GROUPS = B_PER_W // LANES


def _mf_body(users_hbm, items_hbm, utab_hbm, itab_hbm, out_hbm,
             uidx_v, iidx_v, ubuf_v, ibuf_v, out_v, part_v, usem, isem):
    wid = lax.axis_index("s") * 2 + lax.axis_index("c")
    base = wid * B_PER_W

    pltpu.sync_copy(users_hbm.at[pl.ds(base, B_PER_W)], uidx_v)
    pltpu.sync_copy(items_hbm.at[pl.ds(base, B_PER_W)], iidx_v)

    lane_iota = lax.iota(jnp.int32, LANES)

    def group(g, carry):
        uvec = uidx_v[pl.ds(g * LANES, LANES)]
        ivec = iidx_v[pl.ds(g * LANES, LANES)]
        ucps = []
        icps = []
        for j in range(LANES):
            su = uvec[j]
            si = ivec[j]
            ucps.append(pltpu.async_copy(
                utab_hbm.at[pl.ds(su, 1), :], ubuf_v.at[pl.ds(j, 1), :], usem))
            icps.append(pltpu.async_copy(
                itab_hbm.at[pl.ds(si, 1), :], ibuf_v.at[pl.ds(j, 1), :], isem))
        for cp in ucps:
            cp.wait()
        for cp in icps:
            cp.wait()
        for r in range(LANES):
            acc = ubuf_v[r, pl.ds(0, LANES)] * ibuf_v[r, pl.ds(0, LANES)]
            for k in range(1, EMBED_DIM // LANES):
                acc = acc + (ubuf_v[r, pl.ds(k * LANES, LANES)]
                             * ibuf_v[r, pl.ds(k * LANES, LANES)])
            part_v[pl.ds(r * LANES, LANES)] = acc
        tot = plsc.load_gather(part_v, [lane_iota * LANES])
        for l in range(1, LANES):
            tot = tot + plsc.load_gather(part_v, [lane_iota * LANES + l])
        out_v[pl.ds(g * LANES, LANES)] = 1.0 / (1.0 + jnp.exp(-tot))
        return carry

    lax.fori_loop(0, GROUPS, group, 0)

    pltpu.sync_copy(out_v, out_hbm.at[pl.ds(base, B_PER_W)])


@jax.jit
def _mf(users, items, user_table, item_table):
    mesh = plsc.VectorSubcoreMesh(core_axis_name="c", subcore_axis_name="s")
    run = functools.partial(
        pl.kernel,
        mesh=mesh,
        compiler_params=pltpu.CompilerParams(needs_layout_passes=False,
                                             use_tc_tiling_on_sc=True),
        out_type=jax.ShapeDtypeStruct((BATCH,), jnp.float32),
        scratch_types=[
            pltpu.VMEM((B_PER_W,), jnp.int32),
            pltpu.VMEM((B_PER_W,), jnp.int32),
            pltpu.VMEM((LANES, EMBED_DIM), jnp.float32),
            pltpu.VMEM((LANES, EMBED_DIM), jnp.float32),
            pltpu.VMEM((B_PER_W,), jnp.float32),
            pltpu.VMEM((LANES * LANES,), jnp.float32),
            pltpu.SemaphoreType.DMA,
            pltpu.SemaphoreType.DMA,
        ],
    )(_mf_body)
    return run(users, items, user_table, item_table)


def kernel(users, items, user_table, item_table):
    return _mf(users.astype(jnp.int32), items.astype(jnp.int32),
               user_table, item_table)
